# baseline (speedup 1.0000x reference)
import functools

import jax
import jax.numpy as jnp
from jax import lax
from jax.experimental import pallas as pl
from jax.experimental.pallas import tpu as pltpu
from jax.experimental.pallas import tpu_sc as plsc

N = 10000
E = 320000
D_IN = 128
HID = 64
D_OUT = 64

NC = 2
NS = 16
NW = NC * NS
CH = 128
C = 80
E_PAD = NW * C * CH
NROWS = 10112
STRIPE = NROWS // NS
NBUF = 8
PREF = 4
B_TC = 1000

_mesh = plsc.VectorSubcoreMesh(core_axis_name="c", subcore_axis_name="s")


@functools.partial(
    pl.kernel,
    out_type=jax.ShapeDtypeStruct((NC, NROWS, 16), jnp.float32),
    mesh=_mesh,
    scratch_types=[
        pltpu.VMEM((C, CH), jnp.int32),
        pltpu.VMEM((CH, 16), jnp.float32),
        pltpu.VMEM_SHARED((NROWS, 16), jnp.float32),
    ],
    compiler_params=pltpu.CompilerParams(use_tc_tiling_on_sc=False),
)
def _degree_kernel(dst_hbm, zrow_hbm, ones_hbm, out_hbm, dst_v, ones_v, hist_sh):
    cid = lax.axis_index("c")
    sid = lax.axis_index("s")
    wid = cid * NS + sid
    pltpu.sync_copy(zrow_hbm, hist_sh.at[pl.ds(sid * STRIPE, STRIPE)])
    pltpu.sync_copy(dst_hbm.at[wid], dst_v)
    pltpu.sync_copy(ones_hbm, ones_v)
    plsc.subcore_barrier()

    @pl.loop(0, C)
    def _edge_chunk(j):
        pltpu.sync_copy(ones_v, hist_sh.at[dst_v.at[j]], add=True)

    plsc.subcore_barrier()
    pltpu.sync_copy(
        hist_sh.at[pl.ds(sid * STRIPE, STRIPE)],
        out_hbm.at[cid].at[pl.ds(sid * STRIPE, STRIPE)],
    )


@functools.partial(
    pl.kernel,
    out_type=jax.ShapeDtypeStruct((NC, NROWS, HID), jnp.float32),
    mesh=_mesh,
    scratch_types=[
        pltpu.VMEM((C, CH), jnp.int32),
        pltpu.VMEM((C, CH), jnp.int32),
        pltpu.VMEM((NBUF, CH, HID), jnp.float32),
        pltpu.VMEM_SHARED((NROWS, HID), jnp.float32),
        pltpu.SemaphoreType.DMA((NBUF,)),
        pltpu.SemaphoreType.DMA((NBUF,)),
    ],
    compiler_params=pltpu.CompilerParams(use_tc_tiling_on_sc=False),
)
def _scatter_kernel(g_hbm, src_hbm, dst_hbm, zblk_hbm, out_hbm,
                    src_v, dst_v, rows_v, agg_sh, sem_g, sem_s):
    cid = lax.axis_index("c")
    sid = lax.axis_index("s")
    wid = cid * NS + sid
    pltpu.sync_copy(zblk_hbm, agg_sh.at[pl.ds(sid * STRIPE, STRIPE)])
    pltpu.sync_copy(src_hbm.at[wid], src_v)
    pltpu.sync_copy(dst_hbm.at[wid], dst_v)
    plsc.subcore_barrier()

    def _gather(k, b):
        pltpu.async_copy(g_hbm.at[src_v.at[k]], rows_v.at[b], sem_g.at[b])

    def _gather_wait(k, b):
        pltpu.make_async_copy(
            g_hbm.at[src_v.at[k]], rows_v.at[b], sem_g.at[b]
        ).wait()

    def _scatter(k, b):
        pltpu.async_copy(rows_v.at[b], agg_sh.at[dst_v.at[k]], sem_s.at[b],
                         add=True)

    def _scatter_wait(k, b):
        pltpu.make_async_copy(
            rows_v.at[b], agg_sh.at[dst_v.at[k]], sem_s.at[b]
        ).wait()

    for b in range(PREF):
        _gather(b, b)

    @pl.loop(0, C, step=NBUF)
    def _chunk(j):
        for bb in range(NBUF):
            k = j + bb
            _gather_wait(k, bb)
            _scatter(k, bb)
            b2 = (bb + PREF) % NBUF
            k2 = k + PREF

            @pl.when(k2 < C)
            def _prefetch():
                @pl.when(k2 >= NBUF)
                def _free_buf():
                    _scatter_wait(k2 - NBUF, b2)
                _gather(k2, b2)

    for bb in range(NBUF):
        _scatter_wait(C - NBUF + bb, bb)

    plsc.subcore_barrier()
    pltpu.sync_copy(
        agg_sh.at[pl.ds(sid * STRIPE, STRIPE)],
        out_hbm.at[cid].at[pl.ds(sid * STRIPE, STRIPE)],
    )


def _matmul_scale(x, W1, h0, h1):
    def body(x_ref, w_ref, h0_ref, h1_ref, g_ref):
        deg = h0_ref[:, 0:1] + h1_ref[:, 0:1] + 1.0
        dis = lax.rsqrt(deg)
        h = lax.dot_general(
            x_ref[...], w_ref[...], (((1,), (0,)), ((), ())),
            precision=lax.Precision.HIGHEST,
            preferred_element_type=jnp.float32,
        )
        g_ref[...] = h * dis

    return pl.pallas_call(
        body,
        grid=(N // B_TC,),
        in_specs=[
            pl.BlockSpec((B_TC, D_IN), lambda i: (i, 0)),
            pl.BlockSpec((D_IN, HID), lambda i: (0, 0)),
            pl.BlockSpec((B_TC, 16), lambda i: (i, 0)),
            pl.BlockSpec((B_TC, 16), lambda i: (i, 0)),
        ],
        out_specs=pl.BlockSpec((B_TC, HID), lambda i: (i, 0)),
        out_shape=jax.ShapeDtypeStruct((N, HID), jnp.float32),
    )(x, W1, h0, h1)


def _finish(agg0, agg1, g, h0, h1, b1, W2, b2):
    def body(a0_ref, a1_ref, g_ref, h0_ref, h1_ref, b1_ref, w2_ref, b2_ref,
             out_ref, emb_ref):
        deg = h0_ref[:, 0:1] + h1_ref[:, 0:1] + 1.0
        dis = lax.rsqrt(deg)
        s = (a0_ref[...] + a1_ref[...] + g_ref[...]) * dis + b1_ref[...]
        emb = jnp.maximum(s, 0.0)
        emb_ref[...] = emb
        out_ref[...] = lax.dot_general(
            emb, w2_ref[...], (((1,), (0,)), ((), ())),
            precision=lax.Precision.HIGHEST,
            preferred_element_type=jnp.float32,
        ) + b2_ref[...]

    return pl.pallas_call(
        body,
        grid=(N // B_TC,),
        in_specs=[
            pl.BlockSpec((B_TC, HID), lambda i: (i, 0)),
            pl.BlockSpec((B_TC, HID), lambda i: (i, 0)),
            pl.BlockSpec((B_TC, HID), lambda i: (i, 0)),
            pl.BlockSpec((B_TC, 16), lambda i: (i, 0)),
            pl.BlockSpec((B_TC, 16), lambda i: (i, 0)),
            pl.BlockSpec((1, HID), lambda i: (0, 0)),
            pl.BlockSpec((HID, D_OUT), lambda i: (0, 0)),
            pl.BlockSpec((1, D_OUT), lambda i: (0, 0)),
        ],
        out_specs=[
            pl.BlockSpec((B_TC, D_OUT), lambda i: (i, 0)),
            pl.BlockSpec((B_TC, HID), lambda i: (i, 0)),
        ],
        out_shape=[
            jax.ShapeDtypeStruct((N, D_OUT), jnp.float32),
            jax.ShapeDtypeStruct((N, HID), jnp.float32),
        ],
    )(agg0, agg1, g, h0, h1, b1, W2, b2)


def kernel(x, edge_index, W1, b1, W2, b2):
    src = edge_index[0]
    dst = edge_index[1]
    pad = E_PAD - E
    src_p = jnp.concatenate([src, jnp.zeros((pad,), jnp.int32)]).reshape(NW, C, CH)
    dst_p = jnp.concatenate([dst, jnp.full((pad,), N, jnp.int32)]).reshape(NW, C, CH)
    zrow = jnp.zeros((STRIPE, 16), jnp.float32)
    ones_rows = jnp.zeros((CH, 16), jnp.float32).at[:, 0].set(1.0)
    zblk = jnp.zeros((STRIPE, HID), jnp.float32)

    hist = _degree_kernel(dst_p, zrow, ones_rows)
    h0, h1 = hist[0], hist[1]
    g = _matmul_scale(x, W1, h0, h1)
    agg = _scatter_kernel(g, src_p, dst_p, zblk)
    out, emb = _finish(agg[0], agg[1], g, h0, h1,
                       b1.reshape(1, HID), W2, b2.reshape(1, D_OUT))
    return out, emb


# speedup vs baseline: 1.9528x; 1.9528x over previous
import functools

import jax
import jax.numpy as jnp
from jax import lax
from jax.experimental import pallas as pl
from jax.experimental.pallas import tpu as pltpu
from jax.experimental.pallas import tpu_sc as plsc

N = 10000
E = 320000
D_IN = 128
HID = 64
D_OUT = 64

NC = 2
NS = 16
NW = NC * NS
CH = 128
C = 80
E_PAD = NW * C * CH
NROWS = 10112
STRIPE = NROWS // NS
NBUF = 8
PREF = 4
B_TC = 1000

_mesh = plsc.VectorSubcoreMesh(core_axis_name="c", subcore_axis_name="s")


@functools.partial(
    pl.kernel,
    out_type=jax.ShapeDtypeStruct((NC, NROWS, 16), jnp.float32),
    mesh=_mesh,
    scratch_types=[
        pltpu.VMEM((C, CH), jnp.int32),
        pltpu.VMEM((CH, 16), jnp.float32),
        pltpu.VMEM_SHARED((NROWS, 16), jnp.float32),
    ],
    compiler_params=pltpu.CompilerParams(use_tc_tiling_on_sc=False),
)
def _degree_kernel(dst_hbm, zrow_hbm, ones_hbm, out_hbm, dst_v, ones_v, hist_sh):
    cid = lax.axis_index("c")
    sid = lax.axis_index("s")
    wid = cid * NS + sid
    pltpu.sync_copy(zrow_hbm, hist_sh.at[pl.ds(sid * STRIPE, STRIPE)])
    pltpu.sync_copy(dst_hbm.at[wid], dst_v)
    pltpu.sync_copy(ones_hbm, ones_v)
    plsc.subcore_barrier()

    @pl.loop(0, C)
    def _edge_chunk(j):
        pltpu.sync_copy(ones_v, hist_sh.at[dst_v.at[j]], add=True)

    plsc.subcore_barrier()
    pltpu.sync_copy(
        hist_sh.at[pl.ds(sid * STRIPE, STRIPE)],
        out_hbm.at[cid].at[pl.ds(sid * STRIPE, STRIPE)],
    )


@functools.partial(
    pl.kernel,
    out_type=jax.ShapeDtypeStruct((NC, NROWS, HID), jnp.float32),
    mesh=_mesh,
    scratch_types=[
        pltpu.VMEM((C, CH), jnp.int32),
        pltpu.VMEM((C, CH), jnp.int32),
        pltpu.VMEM((NBUF, CH, HID), jnp.float32),
        pltpu.VMEM_SHARED((NROWS, HID), jnp.float32),
        pltpu.SemaphoreType.DMA((NBUF,)),
        pltpu.SemaphoreType.DMA((NBUF,)),
    ],
    compiler_params=pltpu.CompilerParams(use_tc_tiling_on_sc=False),
)
def _scatter_kernel(g_hbm, src_hbm, dst_hbm, zblk_hbm, out_hbm,
                    src_v, dst_v, rows_v, agg_sh, sem_g, sem_s):
    cid = lax.axis_index("c")
    sid = lax.axis_index("s")
    wid = cid * NS + sid
    pltpu.sync_copy(zblk_hbm, agg_sh.at[pl.ds(sid * STRIPE, STRIPE)])
    pltpu.sync_copy(src_hbm.at[wid], src_v)
    pltpu.sync_copy(dst_hbm.at[wid], dst_v)
    plsc.subcore_barrier()

    def _gather(k, b):
        pltpu.async_copy(g_hbm.at[src_v.at[k]], rows_v.at[b], sem_g.at[b])

    def _gather_wait(k, b):
        pltpu.make_async_copy(
            g_hbm.at[src_v.at[k]], rows_v.at[b], sem_g.at[b]
        ).wait()

    def _scatter(k, b):
        pltpu.async_copy(rows_v.at[b], agg_sh.at[dst_v.at[k]], sem_s.at[b],
                         add=True)

    def _scatter_wait(k, b):
        pltpu.make_async_copy(
            rows_v.at[b], agg_sh.at[dst_v.at[k]], sem_s.at[b]
        ).wait()

    for b in range(PREF):
        _gather(b, b)

    @pl.loop(0, C, step=NBUF)
    def _chunk(j):
        for bb in range(NBUF):
            k = j + bb
            _gather_wait(k, bb)
            _scatter(k, bb)
            b2 = (bb + PREF) % NBUF
            k2 = k + PREF

            @pl.when(k2 < C)
            def _prefetch():
                @pl.when(k2 >= NBUF)
                def _free_buf():
                    _scatter_wait(k2 - NBUF, b2)
                _gather(k2, b2)

    for bb in range(NBUF):
        _scatter_wait(C - NBUF + bb, bb)

    plsc.subcore_barrier()
    pltpu.sync_copy(
        agg_sh.at[pl.ds(sid * STRIPE, STRIPE)],
        out_hbm.at[cid].at[pl.ds(sid * STRIPE, STRIPE)],
    )


def _matmul_scale(x, W1, h0, h1):
    def body(x_ref, w_ref, h0_ref, h1_ref, g_ref):
        deg = h0_ref[:, 0:1] + h1_ref[:, 0:1] + 1.0
        dis = lax.rsqrt(deg)
        h = lax.dot_general(
            x_ref[...], w_ref[...], (((1,), (0,)), ((), ())),
            precision=lax.Precision.HIGHEST,
            preferred_element_type=jnp.float32,
        )
        g_ref[...] = h * dis

    return pl.pallas_call(
        body,
        grid=(N // B_TC,),
        in_specs=[
            pl.BlockSpec((B_TC, D_IN), lambda i: (i, 0)),
            pl.BlockSpec((D_IN, HID), lambda i: (0, 0)),
            pl.BlockSpec((B_TC, 16), lambda i: (i, 0)),
            pl.BlockSpec((B_TC, 16), lambda i: (i, 0)),
        ],
        out_specs=pl.BlockSpec((B_TC, HID), lambda i: (i, 0)),
        out_shape=jax.ShapeDtypeStruct((N, HID), jnp.float32),
    )(x, W1, h0, h1)


def _finish(agg0, agg1, g, h0, h1, b1, W2, b2):
    def body(a0_ref, a1_ref, g_ref, h0_ref, h1_ref, b1_ref, w2_ref, b2_ref,
             out_ref, emb_ref):
        deg = h0_ref[:, 0:1] + h1_ref[:, 0:1] + 1.0
        dis = lax.rsqrt(deg)
        s = (a0_ref[...] + a1_ref[...] + g_ref[...]) * dis + b1_ref[...]
        emb = jnp.maximum(s, 0.0)
        emb_ref[...] = emb
        out_ref[...] = lax.dot_general(
            emb, w2_ref[...], (((1,), (0,)), ((), ())),
            precision=lax.Precision.HIGHEST,
            preferred_element_type=jnp.float32,
        ) + b2_ref[...]

    return pl.pallas_call(
        body,
        grid=(N // B_TC,),
        in_specs=[
            pl.BlockSpec((B_TC, HID), lambda i: (i, 0)),
            pl.BlockSpec((B_TC, HID), lambda i: (i, 0)),
            pl.BlockSpec((B_TC, HID), lambda i: (i, 0)),
            pl.BlockSpec((B_TC, 16), lambda i: (i, 0)),
            pl.BlockSpec((B_TC, 16), lambda i: (i, 0)),
            pl.BlockSpec((1, HID), lambda i: (0, 0)),
            pl.BlockSpec((HID, D_OUT), lambda i: (0, 0)),
            pl.BlockSpec((1, D_OUT), lambda i: (0, 0)),
        ],
        out_specs=[
            pl.BlockSpec((B_TC, D_OUT), lambda i: (i, 0)),
            pl.BlockSpec((B_TC, HID), lambda i: (i, 0)),
        ],
        out_shape=[
            jax.ShapeDtypeStruct((N, D_OUT), jnp.float32),
            jax.ShapeDtypeStruct((N, HID), jnp.float32),
        ],
    )(agg0, agg1, g, h0, h1, b1, W2, b2)


def kernel(x, edge_index, W1, b1, W2, b2):
    src = edge_index[0]
    dst = edge_index[1]
    pad = E_PAD - E
    pad_idx = jnp.arange(pad, dtype=jnp.int32)
    src_p = jnp.concatenate([src, pad_idx % N]).reshape(NW, C, CH)
    dst_p = jnp.concatenate([dst, N + pad_idx % (NROWS - N)]).reshape(NW, C, CH)
    zrow = jnp.zeros((STRIPE, 16), jnp.float32)
    ones_rows = jnp.zeros((CH, 16), jnp.float32).at[:, 0].set(1.0)
    zblk = jnp.zeros((STRIPE, HID), jnp.float32)

    hist = _degree_kernel(dst_p, zrow, ones_rows)
    h0, h1 = hist[0], hist[1]
    g = _matmul_scale(x, W1, h0, h1)
    agg = _scatter_kernel(g, src_p, dst_p, zblk)
    out, emb = _finish(agg[0], agg[1], g, h0, h1,
                       b1.reshape(1, HID), W2, b2.reshape(1, D_OUT))
    return out, emb


# speedup vs baseline: 2.1523x; 1.1021x over previous
import functools

import jax
import jax.numpy as jnp
from jax import lax
from jax.experimental import pallas as pl
from jax.experimental.pallas import tpu as pltpu
from jax.experimental.pallas import tpu_sc as plsc

N = 10000
E = 320000
D_IN = 128
HID = 64
D_OUT = 64

NC = 2
NS = 16
NW = NC * NS
CH = 128
NCHUNK = E // CH
C = 80
NROWS = 10112
NDUMP = NROWS - N
STRIPE = NROWS // NS
NBUF = 8
PREF = 4
B_TC = 1000

_mesh = plsc.VectorSubcoreMesh(core_axis_name="c", subcore_axis_name="s")


def _stage_edges(idx_hbm, idx_v, wid, dump_base):
    base = 78 * wid + jnp.minimum(wid, 4)
    pltpu.sync_copy(idx_hbm.at[pl.ds(base, 78)], idx_v.at[pl.ds(0, 78)])

    @pl.when(wid < 4)
    def _real_79th():
        pltpu.sync_copy(idx_hbm.at[pl.ds(base + 78, 1)], idx_v.at[pl.ds(78, 1)])

    lanes = lax.iota(jnp.int32, 16)
    for k in range(CH // 16):
        fill = dump_base + ((lanes + 16 * k) % NDUMP)
        idx_v[79, pl.ds(16 * k, 16)] = fill

        @pl.when(wid >= 4)
        def _fill_78():
            idx_v[78, pl.ds(16 * k, 16)] = fill


@functools.partial(
    pl.kernel,
    out_type=jax.ShapeDtypeStruct((NC, NROWS, 16), jnp.float32),
    mesh=_mesh,
    scratch_types=[
        pltpu.VMEM((C, CH), jnp.int32),
        pltpu.VMEM((CH, 16), jnp.float32),
        pltpu.VMEM_SHARED((NROWS, 16), jnp.float32),
    ],
    compiler_params=pltpu.CompilerParams(use_tc_tiling_on_sc=False),
)
def _degree_kernel(dst_hbm, zrow_hbm, ones_hbm, out_hbm, dst_v, ones_v, hist_sh):
    cid = lax.axis_index("c")
    sid = lax.axis_index("s")
    wid = cid * NS + sid
    pltpu.sync_copy(zrow_hbm, hist_sh.at[pl.ds(sid * STRIPE, STRIPE)])
    pltpu.sync_copy(ones_hbm, ones_v)
    _stage_edges(dst_hbm, dst_v, wid, N)
    plsc.subcore_barrier()

    @pl.loop(0, C)
    def _edge_chunk(j):
        pltpu.sync_copy(ones_v, hist_sh.at[dst_v.at[j]], add=True)

    plsc.subcore_barrier()
    pltpu.sync_copy(
        hist_sh.at[pl.ds(sid * STRIPE, STRIPE)],
        out_hbm.at[cid].at[pl.ds(sid * STRIPE, STRIPE)],
    )


@functools.partial(
    pl.kernel,
    out_type=jax.ShapeDtypeStruct((NC, NROWS, HID), jnp.float32),
    mesh=_mesh,
    scratch_types=[
        pltpu.VMEM((C, CH), jnp.int32),
        pltpu.VMEM((C, CH), jnp.int32),
        pltpu.VMEM((NBUF, CH, HID), jnp.float32),
        pltpu.VMEM_SHARED((NROWS, HID), jnp.float32),
        pltpu.SemaphoreType.DMA((NBUF,)),
        pltpu.SemaphoreType.DMA((NBUF,)),
    ],
    compiler_params=pltpu.CompilerParams(use_tc_tiling_on_sc=False),
)
def _scatter_kernel(g_hbm, src_hbm, dst_hbm, zblk_hbm, out_hbm,
                    src_v, dst_v, rows_v, agg_sh, sem_g, sem_s):
    cid = lax.axis_index("c")
    sid = lax.axis_index("s")
    wid = cid * NS + sid
    pltpu.sync_copy(zblk_hbm, agg_sh.at[pl.ds(sid * STRIPE, STRIPE)])
    _stage_edges(src_hbm, src_v, wid, 0)
    _stage_edges(dst_hbm, dst_v, wid, N)
    plsc.subcore_barrier()

    def _gather(k, b):
        pltpu.async_copy(g_hbm.at[src_v.at[k]], rows_v.at[b], sem_g.at[b])

    def _gather_wait(k, b):
        pltpu.make_async_copy(
            g_hbm.at[src_v.at[k]], rows_v.at[b], sem_g.at[b]
        ).wait()

    def _scatter(k, b):
        pltpu.async_copy(rows_v.at[b], agg_sh.at[dst_v.at[k]], sem_s.at[b],
                         add=True)

    def _scatter_wait(k, b):
        pltpu.make_async_copy(
            rows_v.at[b], agg_sh.at[dst_v.at[k]], sem_s.at[b]
        ).wait()

    for b in range(PREF):
        _gather(b, b)

    @pl.loop(0, C, step=NBUF)
    def _chunk(j):
        for bb in range(NBUF):
            k = j + bb
            _gather_wait(k, bb)
            _scatter(k, bb)
            b2 = (bb + PREF) % NBUF
            k2 = k + PREF

            @pl.when(k2 < C)
            def _prefetch():
                @pl.when(k2 >= NBUF)
                def _free_buf():
                    _scatter_wait(k2 - NBUF, b2)
                _gather(k2, b2)

    for bb in range(NBUF):
        _scatter_wait(C - NBUF + bb, bb)

    plsc.subcore_barrier()
    pltpu.sync_copy(
        agg_sh.at[pl.ds(sid * STRIPE, STRIPE)],
        out_hbm.at[cid].at[pl.ds(sid * STRIPE, STRIPE)],
    )


def _matmul(x, W1):
    def body(x_ref, w_ref, h_ref):
        h_ref[...] = lax.dot_general(
            x_ref[...], w_ref[...], (((1,), (0,)), ((), ())),
            preferred_element_type=jnp.float32,
        )

    return pl.pallas_call(
        body,
        grid=(N // B_TC,),
        in_specs=[
            pl.BlockSpec((B_TC, D_IN), lambda i: (i, 0)),
            pl.BlockSpec((D_IN, HID), lambda i: (0, 0)),
        ],
        out_specs=pl.BlockSpec((B_TC, HID), lambda i: (i, 0)),
        out_shape=jax.ShapeDtypeStruct((N, HID), jnp.float32),
    )(x, W1)


def _scale(h, hist):
    def body(h_ref, h0_ref, h1_ref, g_ref):
        deg = h0_ref[0, :, 0:1] + h1_ref[0, :, 0:1] + 1.0
        g_ref[...] = h_ref[...] * lax.rsqrt(deg)

    return pl.pallas_call(
        body,
        grid=(N // B_TC,),
        in_specs=[
            pl.BlockSpec((B_TC, HID), lambda i: (i, 0)),
            pl.BlockSpec((1, B_TC, 16), lambda i: (0, i, 0)),
            pl.BlockSpec((1, B_TC, 16), lambda i: (1, i, 0)),
        ],
        out_specs=pl.BlockSpec((B_TC, HID), lambda i: (i, 0)),
        out_shape=jax.ShapeDtypeStruct((N, HID), jnp.float32),
    )(h, hist, hist)


def _finish(agg, g, hist, b1, W2, b2):
    def body(a0_ref, a1_ref, g_ref, h0_ref, h1_ref, b1_ref, w2_ref, b2_ref,
             out_ref, emb_ref):
        deg = h0_ref[0, :, 0:1] + h1_ref[0, :, 0:1] + 1.0
        dis = lax.rsqrt(deg)
        s = (a0_ref[0] + a1_ref[0] + g_ref[...]) * dis + b1_ref[...]
        emb = jnp.maximum(s, 0.0)
        emb_ref[...] = emb
        out_ref[...] = lax.dot_general(
            emb, w2_ref[...], (((1,), (0,)), ((), ())),
            preferred_element_type=jnp.float32,
        ) + b2_ref[...]

    return pl.pallas_call(
        body,
        grid=(N // B_TC,),
        in_specs=[
            pl.BlockSpec((1, B_TC, HID), lambda i: (0, i, 0)),
            pl.BlockSpec((1, B_TC, HID), lambda i: (1, i, 0)),
            pl.BlockSpec((B_TC, HID), lambda i: (i, 0)),
            pl.BlockSpec((1, B_TC, 16), lambda i: (0, i, 0)),
            pl.BlockSpec((1, B_TC, 16), lambda i: (1, i, 0)),
            pl.BlockSpec((1, HID), lambda i: (0, 0)),
            pl.BlockSpec((HID, D_OUT), lambda i: (0, 0)),
            pl.BlockSpec((1, D_OUT), lambda i: (0, 0)),
        ],
        out_specs=[
            pl.BlockSpec((B_TC, D_OUT), lambda i: (i, 0)),
            pl.BlockSpec((B_TC, HID), lambda i: (i, 0)),
        ],
        out_shape=[
            jax.ShapeDtypeStruct((N, D_OUT), jnp.float32),
            jax.ShapeDtypeStruct((N, HID), jnp.float32),
        ],
    )(agg, agg, g, hist, hist, b1, W2, b2)


def kernel(x, edge_index, W1, b1, W2, b2):
    src_r = edge_index[0].reshape(NCHUNK, CH)
    dst_r = edge_index[1].reshape(NCHUNK, CH)
    zrow = jnp.zeros((STRIPE, 16), jnp.float32)
    ones_rows = jnp.zeros((CH, 16), jnp.float32).at[:, 0].set(1.0)
    zblk = jnp.zeros((STRIPE, HID), jnp.float32)

    hist = _degree_kernel(dst_r, zrow, ones_rows)
    h = _matmul(x, W1)
    g = _scale(h, hist)
    agg = _scatter_kernel(g, src_r, dst_r, zblk)
    out, emb = _finish(agg, g, hist,
                       b1.reshape(1, HID), W2, b2.reshape(1, D_OUT))
    return out, emb


# speedup vs baseline: 2.5730x; 1.1955x over previous
import functools

import jax
import jax.numpy as jnp
from jax import lax
from jax.experimental import pallas as pl
from jax.experimental.pallas import tpu as pltpu
from jax.experimental.pallas import tpu_sc as plsc

N = 10000
E = 320000
D_IN = 128
HID = 64
D_OUT = 64

NC = 2
NS = 16
NW = NC * NS
CH = 128
NCHUNK = E // CH
C = 80
NROWS = 10112
NDUMP = NROWS - N
STRIPE = NROWS // NS
NBUF = 8
PREF = 4
B_TC = 1000

_mesh = plsc.VectorSubcoreMesh(core_axis_name="c", subcore_axis_name="s")


def _stage_edges(ei_hbm, idx_v, wid):
    base = 78 * wid + jnp.minimum(wid, 4)
    pltpu.sync_copy(ei_hbm.at[pl.ds(base, 78)], idx_v.at[pl.ds(0, 78)])

    @pl.when(wid < 4)
    def _real_79th():
        pltpu.sync_copy(ei_hbm.at[pl.ds(base + 78, 1)], idx_v.at[pl.ds(78, 1)])

    lanes = lax.iota(jnp.int32, 16)
    for k in range(CH // 16):
        src_fill = (lanes + 16 * k) % N
        dst_fill = N + ((lanes + 16 * k) % NDUMP)
        idx_v[79, 0, pl.ds(16 * k, 16)] = src_fill
        idx_v[79, 1, pl.ds(16 * k, 16)] = dst_fill

        @pl.when(wid >= 4)
        def _fill_78():
            idx_v[78, 0, pl.ds(16 * k, 16)] = src_fill
            idx_v[78, 1, pl.ds(16 * k, 16)] = dst_fill


@functools.partial(
    pl.kernel,
    out_type=jax.ShapeDtypeStruct((NC, NROWS, 16), jnp.float32),
    mesh=_mesh,
    scratch_types=[
        pltpu.VMEM((C, 2, CH), jnp.int32),
        pltpu.VMEM((CH, 16), jnp.float32),
        pltpu.VMEM_SHARED((NROWS, 16), jnp.float32),
    ],
    compiler_params=pltpu.CompilerParams(use_tc_tiling_on_sc=False),
)
def _degree_kernel(ei_hbm, zrow_hbm, ones_hbm, out_hbm, ei_v, ones_v, hist_sh):
    cid = lax.axis_index("c")
    sid = lax.axis_index("s")
    wid = cid * NS + sid
    pltpu.sync_copy(zrow_hbm, hist_sh.at[pl.ds(sid * STRIPE, STRIPE)])
    pltpu.sync_copy(ones_hbm, ones_v)
    _stage_edges(ei_hbm, ei_v, wid)
    plsc.subcore_barrier()

    @pl.loop(0, C)
    def _edge_chunk(j):
        pltpu.sync_copy(ones_v, hist_sh.at[ei_v.at[j, 1]], add=True)

    plsc.subcore_barrier()
    pltpu.sync_copy(
        hist_sh.at[pl.ds(sid * STRIPE, STRIPE)],
        out_hbm.at[cid].at[pl.ds(sid * STRIPE, STRIPE)],
    )


@functools.partial(
    pl.kernel,
    out_type=jax.ShapeDtypeStruct((NC, NROWS, HID), jnp.float32),
    mesh=_mesh,
    scratch_types=[
        pltpu.VMEM((C, 2, CH), jnp.int32),
        pltpu.VMEM((NBUF, CH, HID), jnp.float32),
        pltpu.VMEM_SHARED((NROWS, HID), jnp.float32),
        pltpu.SemaphoreType.DMA((NBUF,)),
        pltpu.SemaphoreType.DMA((NBUF,)),
    ],
    compiler_params=pltpu.CompilerParams(use_tc_tiling_on_sc=False),
)
def _scatter_kernel(g_hbm, ei_hbm, zblk_hbm, out_hbm,
                    ei_v, rows_v, agg_sh, sem_g, sem_s):
    cid = lax.axis_index("c")
    sid = lax.axis_index("s")
    wid = cid * NS + sid
    pltpu.sync_copy(zblk_hbm, agg_sh.at[pl.ds(sid * STRIPE, STRIPE)])
    _stage_edges(ei_hbm, ei_v, wid)
    plsc.subcore_barrier()

    def _gather(k, b):
        pltpu.async_copy(g_hbm.at[ei_v.at[k, 0]], rows_v.at[b], sem_g.at[b])

    def _gather_wait(k, b):
        pltpu.make_async_copy(
            g_hbm.at[ei_v.at[k, 0]], rows_v.at[b], sem_g.at[b]
        ).wait()

    def _scatter(k, b):
        pltpu.async_copy(rows_v.at[b], agg_sh.at[ei_v.at[k, 1]], sem_s.at[b],
                         add=True)

    def _scatter_wait(k, b):
        pltpu.make_async_copy(
            rows_v.at[b], agg_sh.at[ei_v.at[k, 1]], sem_s.at[b]
        ).wait()

    for b in range(PREF):
        _gather(b, b)

    @pl.loop(0, C, step=NBUF)
    def _chunk(j):
        for bb in range(NBUF):
            k = j + bb
            _gather_wait(k, bb)
            _scatter(k, bb)
            b2 = (bb + PREF) % NBUF
            k2 = k + PREF

            @pl.when(k2 < C)
            def _prefetch():
                @pl.when(k2 >= NBUF)
                def _free_buf():
                    _scatter_wait(k2 - NBUF, b2)
                _gather(k2, b2)

    for bb in range(NBUF):
        _scatter_wait(C - NBUF + bb, bb)

    plsc.subcore_barrier()
    pltpu.sync_copy(
        agg_sh.at[pl.ds(sid * STRIPE, STRIPE)],
        out_hbm.at[cid].at[pl.ds(sid * STRIPE, STRIPE)],
    )


def _matmul(x, W1):
    def body(x_ref, w_ref, h_ref):
        h_ref[...] = lax.dot_general(
            x_ref[...], w_ref[...], (((1,), (0,)), ((), ())),
            preferred_element_type=jnp.float32,
        )

    return pl.pallas_call(
        body,
        grid=(N // B_TC,),
        in_specs=[
            pl.BlockSpec((B_TC, D_IN), lambda i: (i, 0)),
            pl.BlockSpec((D_IN, HID), lambda i: (0, 0)),
        ],
        out_specs=pl.BlockSpec((B_TC, HID), lambda i: (i, 0)),
        out_shape=jax.ShapeDtypeStruct((N, HID), jnp.float32),
    )(x, W1)


def _scale(h, hist):
    def body(h_ref, h0_ref, h1_ref, g_ref):
        deg = h0_ref[0, :, 0:1] + h1_ref[0, :, 0:1] + 1.0
        g_ref[...] = h_ref[...] * lax.rsqrt(deg)

    return pl.pallas_call(
        body,
        grid=(N // B_TC,),
        in_specs=[
            pl.BlockSpec((B_TC, HID), lambda i: (i, 0)),
            pl.BlockSpec((1, B_TC, 16), lambda i: (0, i, 0)),
            pl.BlockSpec((1, B_TC, 16), lambda i: (1, i, 0)),
        ],
        out_specs=pl.BlockSpec((B_TC, HID), lambda i: (i, 0)),
        out_shape=jax.ShapeDtypeStruct((N, HID), jnp.float32),
    )(h, hist, hist)


def _finish(agg, g, hist, b1, W2, b2):
    def body(a_ref, g_ref, h_ref, b1_ref, w2_ref, b2_ref, out_ref, emb_ref):
        deg = h_ref[0, :N, 0:1] + h_ref[1, :N, 0:1] + 1.0
        dis = lax.rsqrt(deg)
        s = (a_ref[0, :N] + a_ref[1, :N] + g_ref[...]) * dis + b1_ref[...]
        emb_t = lax.transpose(jnp.maximum(s, 0.0), (1, 0))
        emb_ref[...] = emb_t
        out_ref[...] = lax.dot_general(
            w2_ref[...], emb_t, (((0,), (0,)), ((), ())),
            preferred_element_type=jnp.float32,
        ) + b2_ref[...]

    return pl.pallas_call(
        body,
        out_shape=[
            jax.ShapeDtypeStruct((D_OUT, N), jnp.float32),
            jax.ShapeDtypeStruct((HID, N), jnp.float32),
        ],
    )(agg, g, hist, b1, W2, b2)


def kernel(x, edge_index, W1, b1, W2, b2):
    ei3 = edge_index.reshape(2, NCHUNK, CH).transpose(1, 0, 2)
    zrow = jnp.zeros((STRIPE, 16), jnp.float32)
    ones_rows = jnp.zeros((CH, 16), jnp.float32).at[:, 0].set(1.0)
    zblk = jnp.zeros((STRIPE, HID), jnp.float32)

    hist = _degree_kernel(ei3, zrow, ones_rows)
    h = _matmul(x, W1)
    g = _scale(h, hist)
    agg = _scatter_kernel(g, ei3, zblk)
    out_t, emb_t = _finish(agg, g, hist,
                           b1.reshape(1, HID), W2, b2.reshape(D_OUT, 1))

    return out_t.T, emb_t.T


# speedup vs baseline: 2.9475x; 1.1455x over previous
import functools

import jax
import jax.numpy as jnp
from jax import lax
from jax.experimental import pallas as pl
from jax.experimental.pallas import tpu as pltpu
from jax.experimental.pallas import tpu_sc as plsc

N = 10000
E = 320000
D_IN = 128
HID = 64
D_OUT = 64

NC = 2
NS = 16
NW = NC * NS
CH = 128
NCHUNK = E // CH
C = 80
NROWS = 10112
NDUMP = NROWS - N
STRIPE = NROWS // NS
NBUF = 8
PREF = 4
B_TC = 1000

_mesh = plsc.VectorSubcoreMesh(core_axis_name="c", subcore_axis_name="s")


def _stage_edges(ei_hbm, idx_v, wid):
    base = 78 * wid + jnp.minimum(wid, 4)
    pltpu.sync_copy(ei_hbm.at[pl.ds(base, 78)], idx_v.at[pl.ds(0, 78)])

    @pl.when(wid < 4)
    def _real_79th():
        pltpu.sync_copy(ei_hbm.at[pl.ds(base + 78, 1)], idx_v.at[pl.ds(78, 1)])

    lanes = lax.iota(jnp.int32, 16)
    for k in range(CH // 16):
        src_fill = (lanes + 16 * k) % N
        dst_fill = N + ((lanes + 16 * k) % NDUMP)
        idx_v[79, 0, pl.ds(16 * k, 16)] = src_fill
        idx_v[79, 1, pl.ds(16 * k, 16)] = dst_fill

        @pl.when(wid >= 4)
        def _fill_78():
            idx_v[78, 0, pl.ds(16 * k, 16)] = src_fill
            idx_v[78, 1, pl.ds(16 * k, 16)] = dst_fill


@functools.partial(
    pl.kernel,
    out_type=jax.ShapeDtypeStruct((NW, NROWS), jnp.float32),
    mesh=_mesh,
    scratch_types=[
        pltpu.VMEM((C * 2 * CH,), jnp.int32),
        pltpu.VMEM((NROWS,), jnp.float32),
    ],
    compiler_params=pltpu.CompilerParams(
        use_tc_tiling_on_sc=False, needs_layout_passes=False),
)
def _degree_kernel(ei_hbm, out_hbm, ei_v, hist_v):
    cid = lax.axis_index("c")
    sid = lax.axis_index("s")
    wid = cid * NS + sid
    base = 78 * wid + jnp.minimum(wid, 4)
    pltpu.sync_copy(ei_hbm.at[pl.ds(base * 2 * CH, 78 * 2 * CH)],
                    ei_v.at[pl.ds(0, 78 * 2 * CH)])

    @pl.when(wid < 4)
    def _real_79th():
        pltpu.sync_copy(ei_hbm.at[pl.ds((base + 78) * 2 * CH, 2 * CH)],
                        ei_v.at[pl.ds(78 * 2 * CH, 2 * CH)])

    lanes = lax.iota(jnp.int32, 16)
    for k in range(CH // 16):
        dst_fill = N + ((lanes + 16 * k) % NDUMP)
        ei_v[pl.ds(79 * 2 * CH + CH + 16 * k, 16)] = dst_fill

        @pl.when(wid >= 4)
        def _fill_78():
            ei_v[pl.ds(78 * 2 * CH + CH + 16 * k, 16)] = dst_fill

    zeros = jnp.zeros((16,), jnp.float32)

    @pl.loop(0, NROWS // 16)
    def _zero(j):
        hist_v[pl.ds(16 * j, 16)] = zeros

    ones = jnp.ones((16,), jnp.float32)

    @pl.loop(0, C)
    def _edge_chunk(j):
        for k in range(CH // 16):
            idx = ei_v[pl.ds(j * 2 * CH + CH + 16 * k, 16)]
            plsc.addupdate_scatter(hist_v, [idx], ones)

    pltpu.sync_copy(hist_v, out_hbm.at[wid])


@functools.partial(
    pl.kernel,
    out_type=jax.ShapeDtypeStruct((NC, NROWS, HID), jnp.float32),
    mesh=_mesh,
    scratch_types=[
        pltpu.VMEM((C, 2, CH), jnp.int32),
        pltpu.VMEM((NBUF, CH, HID), jnp.float32),
        pltpu.VMEM_SHARED((NROWS, HID), jnp.float32),
        pltpu.SemaphoreType.DMA((NBUF,)),
        pltpu.SemaphoreType.DMA((NBUF,)),
    ],
    compiler_params=pltpu.CompilerParams(use_tc_tiling_on_sc=False),
)
def _scatter_kernel(g_hbm, ei_hbm, zblk_hbm, out_hbm,
                    ei_v, rows_v, agg_sh, sem_g, sem_s):
    cid = lax.axis_index("c")
    sid = lax.axis_index("s")
    wid = cid * NS + sid
    pltpu.sync_copy(zblk_hbm, agg_sh.at[pl.ds(sid * STRIPE, STRIPE)])
    _stage_edges(ei_hbm, ei_v, wid)
    plsc.subcore_barrier()

    def _gather(k, b):
        pltpu.async_copy(g_hbm.at[ei_v.at[k, 0]], rows_v.at[b], sem_g.at[b])

    def _gather_wait(k, b):
        pltpu.make_async_copy(
            g_hbm.at[ei_v.at[k, 0]], rows_v.at[b], sem_g.at[b]
        ).wait()

    def _scatter(k, b):
        pltpu.async_copy(rows_v.at[b], agg_sh.at[ei_v.at[k, 1]], sem_s.at[b],
                         add=True)

    def _scatter_wait(k, b):
        pltpu.make_async_copy(
            rows_v.at[b], agg_sh.at[ei_v.at[k, 1]], sem_s.at[b]
        ).wait()

    for b in range(PREF):
        _gather(b, b)

    @pl.loop(0, C, step=NBUF)
    def _chunk(j):
        for bb in range(NBUF):
            k = j + bb
            _gather_wait(k, bb)
            _scatter(k, bb)
            b2 = (bb + PREF) % NBUF
            k2 = k + PREF

            @pl.when(k2 < C)
            def _prefetch():
                @pl.when(k2 >= NBUF)
                def _free_buf():
                    _scatter_wait(k2 - NBUF, b2)
                _gather(k2, b2)

    for bb in range(NBUF):
        _scatter_wait(C - NBUF + bb, bb)

    plsc.subcore_barrier()
    pltpu.sync_copy(
        agg_sh.at[pl.ds(sid * STRIPE, STRIPE)],
        out_hbm.at[cid].at[pl.ds(sid * STRIPE, STRIPE)],
    )


def _matmul(x, W1):
    def body(x_ref, w_ref, h_ref):
        h_ref[...] = lax.dot_general(
            x_ref[...], w_ref[...], (((1,), (0,)), ((), ())),
            preferred_element_type=jnp.float32,
        )

    return pl.pallas_call(
        body,
        grid=(N // B_TC,),
        in_specs=[
            pl.BlockSpec((B_TC, D_IN), lambda i: (i, 0)),
            pl.BlockSpec((D_IN, HID), lambda i: (0, 0)),
        ],
        out_specs=pl.BlockSpec((B_TC, HID), lambda i: (i, 0)),
        out_shape=jax.ShapeDtypeStruct((N, HID), jnp.float32),
    )(x, W1)


def _dis_column(ht_ref):
    ht = lax.transpose(ht_ref[...], (1, 0))
    deg = jnp.sum(ht, axis=1, keepdims=True)[:N] + 1.0
    return lax.rsqrt(deg)


def _scale(h, hist):
    def body(h_ref, ht_ref, g_ref):
        g_ref[...] = h_ref[...] * _dis_column(ht_ref)

    return pl.pallas_call(
        body,
        out_shape=jax.ShapeDtypeStruct((N, HID), jnp.float32),
    )(h, hist)


def _finish(agg, g, hist, b1, W2, b2):
    def body(a_ref, g_ref, ht_ref, b1_ref, w2_ref, b2_ref, out_ref, emb_ref):
        dis = _dis_column(ht_ref)
        s = (a_ref[0, :N] + a_ref[1, :N] + g_ref[...]) * dis + b1_ref[...]
        emb_t = lax.transpose(jnp.maximum(s, 0.0), (1, 0))
        emb_ref[...] = emb_t
        out_ref[...] = lax.dot_general(
            w2_ref[...], emb_t, (((0,), (0,)), ((), ())),
            preferred_element_type=jnp.float32,
        ) + b2_ref[...]

    return pl.pallas_call(
        body,
        out_shape=[
            jax.ShapeDtypeStruct((D_OUT, N), jnp.float32),
            jax.ShapeDtypeStruct((HID, N), jnp.float32),
        ],
    )(agg, g, hist, b1, W2, b2)


def kernel(x, edge_index, W1, b1, W2, b2):
    ei3 = edge_index.reshape(2, NCHUNK, CH).transpose(1, 0, 2)
    zblk = jnp.zeros((STRIPE, HID), jnp.float32)

    hist = _degree_kernel(ei3.reshape(-1))
    h = _matmul(x, W1)
    g = _scale(h, hist)
    agg = _scatter_kernel(g, ei3, zblk)
    out_t, emb_t = _finish(agg, g, hist,
                           b1.reshape(1, HID), W2, b2.reshape(D_OUT, 1))
    return out_t.T, emb_t.T
